# baseline (speedup 1.0000x reference)
import jax
import jax.numpy as jnp
from jax.experimental import pallas as pl
from jax.experimental.pallas import tpu as pltpu


def _fused_body(x_ref, w_ref, b_ref, p_ref, o_ref, wb_ref, pm1_ref):
    @pl.when(pl.program_id(0) == 0)
    def _prep():
        wb_ref[...] = w_ref[...].astype(jnp.bfloat16)
        pm1_ref[...] = p_ref[...].astype(jnp.bfloat16) - jnp.bfloat16(1.0)

    feats = jnp.dot(
        x_ref[...].astype(jnp.bfloat16),
        wb_ref[...],
        preferred_element_type=jnp.float32,
    )
    feats = feats + b_ref[...]
    f = (feats > 0.5).astype(jnp.bfloat16)
    o_ref[...] = jax.lax.dot_general(
        f, pm1_ref[...], (((1,), (1,)), ((), ())),
        preferred_element_type=jnp.float32,
    )


def kernel(x, W, b, predicate_matrix):
    B, D = x.shape
    F = W.shape[1]
    C = predicate_matrix.shape[0]
    bm = 512 if B % 512 == 0 else B
    b2 = b.reshape(1, F)
    return pl.pallas_call(
        _fused_body,
        grid=(B // bm,),
        in_specs=[
            pl.BlockSpec((bm, D), lambda i: (i, 0)),
            pl.BlockSpec((D, F), lambda i: (0, 0)),
            pl.BlockSpec((1, F), lambda i: (0, 0)),
            pl.BlockSpec((C, F), lambda i: (0, 0)),
        ],
        out_specs=pl.BlockSpec((bm, C), lambda i: (i, 0)),
        out_shape=jax.ShapeDtypeStruct((B, C), jnp.float32),
        scratch_shapes=[
            pltpu.VMEM((D, F), jnp.bfloat16),
            pltpu.VMEM((C, F), jnp.bfloat16),
        ],
    )(x, W, b2, predicate_matrix)


# speedup vs baseline: 1.0644x; 1.0644x over previous
import jax
import jax.numpy as jnp
from jax.experimental import pallas as pl
from jax.experimental.pallas import tpu as pltpu


def _fused_body(x_ref, w_ref, b_ref, p_ref, o_ref):
    feats = jnp.dot(x_ref[...], w_ref[...], preferred_element_type=jnp.float32)
    feats = feats + b_ref[...]
    f = (feats > 0.5).astype(jnp.float32)
    pm1 = p_ref[...] - 1.0
    o_ref[...] = jax.lax.dot_general(
        f, pm1, (((1,), (1,)), ((), ())),
        preferred_element_type=jnp.float32,
    )


def kernel(x, W, b, predicate_matrix):
    B, D = x.shape
    F = W.shape[1]
    C = predicate_matrix.shape[0]
    bm = 512 if B % 512 == 0 else B
    b2 = b.reshape(1, F)
    return pl.pallas_call(
        _fused_body,
        grid=(B // bm,),
        in_specs=[
            pl.BlockSpec((bm, D), lambda i: (i, 0)),
            pl.BlockSpec((D, F), lambda i: (0, 0)),
            pl.BlockSpec((1, F), lambda i: (0, 0)),
            pl.BlockSpec((C, F), lambda i: (0, 0)),
        ],
        out_specs=pl.BlockSpec((bm, C), lambda i: (i, 0)),
        out_shape=jax.ShapeDtypeStruct((B, C), jnp.float32),
        compiler_params=pltpu.CompilerParams(
            dimension_semantics=("parallel",),
        ),
    )(x, W, b2, predicate_matrix)
